# baseline (speedup 1.0000x reference)
import functools

import jax
import jax.numpy as jnp
from jax import lax
from jax.experimental import pallas as pl
from jax.experimental.pallas import tpu as pltpu
from jax.experimental.pallas import tpu_sc as plsc

VOCAB = 1000000
D = 64
NC, NS = 2, 16
NW = NC * NS
N_FULL = 244
N_BLK = VOCAB // 128

_mesh = plsc.VectorSubcoreMesh(core_axis_name="c", subcore_axis_name="s")


@functools.partial(
    pl.kernel,
    out_type=jax.ShapeDtypeStruct((VOCAB // 2, 128), jnp.float32),
    mesh=_mesh,
    compiler_params=pltpu.CompilerParams(needs_layout_passes=False),
    scratch_types=[
        pltpu.VMEM((D, 128), jnp.float32),
        pltpu.VMEM((D, 128), jnp.float32),
        pltpu.VMEM((D, 129), jnp.float32),
        pltpu.VMEM((D, 129), jnp.float32),
        pltpu.VMEM((D, D), jnp.float32),
        pltpu.SemaphoreType.DMA,
        pltpu.SemaphoreType.DMA,
        pltpu.SemaphoreType.DMA,
        pltpu.SemaphoreType.DMA,
    ],
)
def _table_rows(tT_hbm, tail_hbm, out_hbm, s0, s1, p0, p1, s2,
                semg0, semg1, sems0, sems1):
    w = lax.axis_index("s") * NC + lax.axis_index("c")

    sbufs = (s0, s1)
    pbufs = (p0, p1)
    semgs = (semg0, semg1)
    semss = (sems0, sems1)

    iota16 = lax.iota(jnp.int32, 16)
    rowv = [(iota16 + 16 * m) // 2 for m in range(8)]
    colv = (iota16 % 2) * 64

    def fire_load(c, q):
        pltpu.async_copy(tT_hbm.at[:, pl.ds(c * 128, 128)], sbufs[q], semgs[q])

    def wait_load(c, q):
        pltpu.make_async_copy(
            tT_hbm.at[:, pl.ds(c * 128, 128)], sbufs[q], semgs[q]).wait()

    def transpose(q, n_m, src=None):
        s, p = (src if src is not None else sbufs[q]), pbufs[q]
        zeros = jnp.zeros((16,), jnp.int32)

        def body(d, carry):
            cv = colv + (zeros + d)
            for m in range(n_m):
                v = s[d, pl.ds(16 * m, 16)]
                plsc.store_scatter(p, [rowv[m], cv], v)
            return carry

        lax.fori_loop(0, D, body, 0)

    def fire_store(c, q, nrows):
        pltpu.async_copy(
            pbufs[q].at[pl.ds(0, nrows), pl.ds(0, 128)],
            out_hbm.at[pl.ds(c * 64, nrows)], semss[q])

    def wait_store(c, q, nrows):
        pltpu.make_async_copy(
            pbufs[q].at[pl.ds(0, nrows), pl.ds(0, 128)],
            out_hbm.at[pl.ds(c * 64, nrows)], semss[q]).wait()

    fire_load(w, 0)
    fire_load(w + 32, 1)

    def step2(i2, carry):
        for half in range(2):
            i = 2 * i2 + half
            q = half
            c = w + 32 * i
            wait_load(c, q)

            @pl.when(i >= 2)
            def _():
                wait_store(w + 32 * (i - 2), q, D)

            transpose(q, 8)
            fire_store(c, q, D)

            @pl.when(i + 2 < N_FULL)
            def _():
                fire_load(w + 32 * (i + 2), q)

        return carry

    lax.fori_loop(0, N_FULL // 2, step2, 0)
    wait_store(w + 32 * (N_FULL - 2), 0, D)
    wait_store(w + 32 * (N_FULL - 1), 1, D)

    @pl.when(w < 4)
    def _():
        c = 7808 + w
        pltpu.sync_copy(tT_hbm.at[:, pl.ds(c * 128, 128)], s0)
        transpose(0, 8)
        fire_store(c, 0, D)
        wait_store(c, 0, D)

    @pl.when(w == 4)
    def _():
        pltpu.sync_copy(tail_hbm, s2)
        transpose(0, 4, src=s2)
        fire_store(N_BLK, 0, 32)
        wait_store(N_BLK, 0, 32)


B_TOTAL = 4096 * 200
ROW = 128
N_ROWS = B_TOTAL // ROW
ROWS_PER_W = N_ROWS // NW
CH_ROWS = 4
CHUNK = CH_ROWS * ROW
N_CH = ROWS_PER_W // CH_ROWS


@functools.partial(
    pl.kernel,
    out_type=jax.ShapeDtypeStruct((B_TOTAL, D), jnp.float32),
    mesh=_mesh,
    compiler_params=pltpu.CompilerParams(
        use_tc_tiling_on_sc=False, needs_layout_passes=False),
    scratch_types=[
        pltpu.VMEM((ROWS_PER_W, ROW), jnp.int32),
        pltpu.VMEM((CHUNK, D), jnp.float32),
        pltpu.VMEM((CHUNK, D), jnp.float32),
        pltpu.SemaphoreType.DMA,
        pltpu.SemaphoreType.DMA,
        pltpu.SemaphoreType.DMA,
        pltpu.SemaphoreType.DMA,
    ],
)
def _embed_gather(idx_hbm, tbl_hbm, out_hbm, idx_all, rows_a, rows_b,
                  semg_a, semg_b, sems_a, sems_b):
    wid = lax.axis_index("s") * NC + lax.axis_index("c")
    irow0 = wid * ROWS_PER_W
    orow0 = wid * ROWS_PER_W * ROW

    pltpu.sync_copy(idx_hbm.at[pl.ds(irow0, ROWS_PER_W)], idx_all)

    def fire_gather(c, buf, sem):
        for j in range(CH_ROWS):
            pltpu.async_copy(
                tbl_hbm.at[idx_all.at[c * CH_ROWS + j]],
                buf.at[pl.ds(j * ROW, ROW)], sem)

    def wait_gather(c, buf, sem):
        for j in range(CH_ROWS):
            pltpu.make_async_copy(
                tbl_hbm.at[idx_all.at[c * CH_ROWS + j]],
                buf.at[pl.ds(j * ROW, ROW)], sem).wait()

    def fire_store(c, buf, sem):
        pltpu.async_copy(buf, out_hbm.at[pl.ds(orow0 + c * CHUNK, CHUNK)], sem)

    def wait_store(c, buf, sem):
        pltpu.make_async_copy(
            buf, out_hbm.at[pl.ds(orow0 + c * CHUNK, CHUNK)], sem).wait()

    fire_gather(0, rows_a, semg_a)
    fire_gather(1, rows_b, semg_b)

    def step(k, carry):
        a = 2 * k
        b = 2 * k + 1
        wait_gather(a, rows_a, semg_a)
        fire_store(a, rows_a, sems_a)
        wait_gather(b, rows_b, semg_b)
        fire_store(b, rows_b, sems_b)

        @pl.when(k < N_CH // 2 - 1)
        def _():
            wait_store(a, rows_a, sems_a)
            fire_gather(a + 2, rows_a, semg_a)
            wait_store(b, rows_b, sems_b)
            fire_gather(b + 2, rows_b, semg_b)

        return carry

    lax.fori_loop(0, N_CH // 2, step, 0)
    wait_store(N_CH - 2, rows_a, sems_a)
    wait_store(N_CH - 1, rows_b, sems_b)


def kernel(x, table):
    tail = table[N_BLK * 128:].T
    tbl2 = _table_rows(table.T, tail)
    tbl = tbl2.reshape(VOCAB, D)
    idx = x.reshape(N_ROWS, ROW)
    out = _embed_gather(idx, tbl)
    out2 = lax.optimization_barrier(out.reshape(B_TOTAL // 2, 2 * D))
    return out2.reshape(4096, 200, D)


# speedup vs baseline: 1.5050x; 1.5050x over previous
import functools

import jax
import jax.numpy as jnp
from jax import lax
from jax.experimental import pallas as pl
from jax.experimental.pallas import tpu as pltpu
from jax.experimental.pallas import tpu_sc as plsc

VOCAB = 1000000
D = 64
B_TOTAL = 4096 * 200
ROW = 128
N_ROWS = B_TOTAL // ROW
NC, NS = 2, 16
NW = NC * NS
ROWS_PER_W = N_ROWS // NW
CH_ROWS = 4
CHUNK = CH_ROWS * ROW
N_CH = ROWS_PER_W // CH_ROWS

_mesh = plsc.VectorSubcoreMesh(core_axis_name="c", subcore_axis_name="s")


@functools.partial(
    pl.kernel,
    out_type=jax.ShapeDtypeStruct((B_TOTAL, D), jnp.float32),
    mesh=_mesh,
    compiler_params=pltpu.CompilerParams(use_tc_tiling_on_sc=False),
    scratch_types=[
        pltpu.VMEM((ROWS_PER_W, ROW), jnp.int32),
        pltpu.VMEM((CHUNK, D), jnp.float32),
        pltpu.VMEM((CHUNK, D), jnp.float32),
        pltpu.SemaphoreType.DMA,
        pltpu.SemaphoreType.DMA,
        pltpu.SemaphoreType.DMA,
        pltpu.SemaphoreType.DMA,
    ],
)
def _embed_gather(idx_hbm, table_hbm, out_hbm, idx_all, rows_a, rows_b,
                  semg_a, semg_b, sems_a, sems_b):
    wid = lax.axis_index("s") * NC + lax.axis_index("c")
    irow0 = wid * ROWS_PER_W
    orow0 = wid * ROWS_PER_W * ROW

    pltpu.sync_copy(idx_hbm.at[pl.ds(irow0, ROWS_PER_W)], idx_all)

    def fire_gather(c, buf, sem):
        for j in range(CH_ROWS):
            pltpu.async_copy(
                table_hbm.at[idx_all.at[c * CH_ROWS + j]],
                buf.at[pl.ds(j * ROW, ROW)],
                sem,
            )

    def wait_gather(c, buf, sem):
        for j in range(CH_ROWS):
            pltpu.make_async_copy(
                table_hbm.at[idx_all.at[c * CH_ROWS + j]],
                buf.at[pl.ds(j * ROW, ROW)],
                sem,
            ).wait()

    def fire_store(c, buf, sem):
        pltpu.async_copy(buf, out_hbm.at[pl.ds(orow0 + c * CHUNK, CHUNK)], sem)

    def wait_store(c, buf, sem):
        pltpu.make_async_copy(
            buf, out_hbm.at[pl.ds(orow0 + c * CHUNK, CHUNK)], sem
        ).wait()

    fire_gather(0, rows_a, semg_a)
    fire_gather(1, rows_b, semg_b)

    def step(k, carry):
        a = 2 * k
        b = 2 * k + 1
        wait_gather(a, rows_a, semg_a)
        fire_store(a, rows_a, sems_a)
        wait_gather(b, rows_b, semg_b)
        fire_store(b, rows_b, sems_b)

        @pl.when(k < N_CH // 2 - 1)
        def _refill():
            wait_store(a, rows_a, sems_a)
            fire_gather(a + 2, rows_a, semg_a)
            wait_store(b, rows_b, sems_b)
            fire_gather(b + 2, rows_b, semg_b)

        return carry

    lax.fori_loop(0, N_CH // 2, step, 0)

    wait_store(N_CH - 2, rows_a, sems_a)
    wait_store(N_CH - 1, rows_b, sems_b)


def kernel(x, table):
    idx = x.reshape(N_ROWS, ROW)
    out = _embed_gather(idx, table)
    return out.reshape(4096, 200, D)
